# baseline (speedup 1.0000x reference)
import jax
import jax.numpy as jnp
from jax import lax
from jax.experimental import pallas as pl
from jax.experimental.pallas import tpu as pltpu
from jax.experimental.pallas import tpu_sc as plsc

M = 64
S = 12
R = 3
D = 1024
T = M * R * S
B = 4

TRIPLES_PER_TILE = 8
TILE_T = TRIPLES_PER_TILE * R * S

NC, NS = 2, 16
NW = NC * NS
TPW = (B * T) // NW


def _tc_body(x_ref, wt_ref, wr_ref, wk_ref, lat_ref):
    x = x_ref[...]
    wt = wt_ref[...]
    wr = wr_ref[...]
    wk = wk_ref[...]
    p36 = (jnp.repeat(wr, S, axis=0) + jnp.tile(wk, (R, 1)))
    pos = (wt[:, None, :] + p36[None, :, :]).reshape(TILE_T, D)
    lat_ref[...] = x + pos[None]


CPR = 48
NCHUNK = TPW // CPR


def _sc_copy_body(x_hbm, cp_hbm, buf0, buf1, is0, is1, os0, os1):
    cid = lax.axis_index("c")
    sid = lax.axis_index("s")
    wid = sid * NC + cid
    b = wid // 8
    t0 = (wid % 8) * TPW

    bufs = (buf0, buf1)
    isems = (is0, is1)
    osems = (os0, os1)

    def in_cp(j):
        return pltpu.make_async_copy(
            x_hbm.at[b, pl.ds(t0 + j * CPR, CPR)], bufs[j % 2], isems[j % 2])

    def out_cp(j):
        return pltpu.make_async_copy(
            bufs[j % 2], cp_hbm.at[b, pl.ds(t0 + j * CPR, CPR)], osems[j % 2])

    in_cp(0).start()
    for j in range(NCHUNK):
        in_cp(j).wait()
        out_cp(j).start()
        if j + 1 < NCHUNK:
            if j >= 1:
                out_cp(j - 1).wait()
            in_cp(j + 1).start()
    out_cp(NCHUNK - 2).wait()
    out_cp(NCHUNK - 1).wait()


def kernel(token_embeds, pad_mask, W_triple, W_role, W_tokpos):
    out_sds = jax.ShapeDtypeStruct((B, T, D), token_embeds.dtype)

    latent = pl.pallas_call(
        _tc_body,
        grid=(T // TILE_T,),
        in_specs=[
            pl.BlockSpec((B, TILE_T, D), lambda t: (0, t, 0)),
            pl.BlockSpec((TRIPLES_PER_TILE, D), lambda t: (t, 0)),
            pl.BlockSpec((R, D), lambda t: (0, 0)),
            pl.BlockSpec((S, D), lambda t: (0, 0)),
        ],
        out_specs=pl.BlockSpec((B, TILE_T, D), lambda t: (0, t, 0)),
        out_shape=out_sds,
    )(token_embeds, W_triple, W_role, W_tokpos)

    copy = pl.kernel(
        _sc_copy_body,
        out_type=out_sds,
        mesh=plsc.VectorSubcoreMesh(core_axis_name="c", subcore_axis_name="s"),
        scratch_types=[
            pltpu.VMEM((CPR, D), jnp.float32),
            pltpu.VMEM((CPR, D), jnp.float32),
            pltpu.SemaphoreType.DMA,
            pltpu.SemaphoreType.DMA,
            pltpu.SemaphoreType.DMA,
            pltpu.SemaphoreType.DMA,
        ],
    )(token_embeds)

    return (latent, copy)


# speedup vs baseline: 1.8441x; 1.8441x over previous
import jax
import jax.numpy as jnp
from jax.experimental import pallas as pl
from jax.experimental.pallas import tpu as pltpu

M = 64
S = 12
R = 3
D = 1024
T = M * R * S

TRIPLES_PER_TILE = 8
TILE_T = TRIPLES_PER_TILE * R * S


def _body(x_ref, wt_ref, wr_ref, wk_ref, lat_ref, cp_ref):
    x = x_ref[...]
    wt = wt_ref[...]
    wr = wr_ref[...]
    wk = wk_ref[...]
    p36 = (jnp.repeat(wr, S, axis=0) + jnp.tile(wk, (R, 1)))
    pos = (wt[:, None, :] + p36[None, :, :]).reshape(TILE_T, D)
    lat_ref[...] = x + pos[None]
    cp_ref[...] = x


def kernel(token_embeds, pad_mask, W_triple, W_role, W_tokpos):
    B = token_embeds.shape[0]
    grid = (T // TILE_T,)
    out_sds = jax.ShapeDtypeStruct((B, T, D), token_embeds.dtype)
    latent, copy = pl.pallas_call(
        _body,
        grid=grid,
        in_specs=[
            pl.BlockSpec((B, TILE_T, D), lambda t: (0, t, 0)),
            pl.BlockSpec((TRIPLES_PER_TILE, D), lambda t: (t, 0)),
            pl.BlockSpec((R, D), lambda t: (0, 0)),
            pl.BlockSpec((S, D), lambda t: (0, 0)),
        ],
        out_specs=[
            pl.BlockSpec((B, TILE_T, D), lambda t: (0, t, 0)),
            pl.BlockSpec((B, TILE_T, D), lambda t: (0, t, 0)),
        ],
        out_shape=[out_sds, out_sds],
        compiler_params=pltpu.CompilerParams(
            dimension_semantics=("parallel",),
        ),
    )(token_embeds, W_triple, W_role, W_tokpos)
    return (latent, copy)
